# baseline (speedup 1.0000x reference)
import functools

import jax
import jax.numpy as jnp
from jax import lax
from jax.experimental import pallas as pl
from jax.experimental.pallas import tpu as pltpu
from jax.experimental.pallas import tpu_sc as plsc

_NUM_EMBEDDINGS = 8192
_EMBEDDING_DIM = 256
_COMMITMENT_COST = 0.25

_TM = 1024
_TE = 1024
_B1 = 2736
_B2 = 5472


def _fused_body(flat_ref, fsq_ref, embed_ref,
                idx_out, loss_out, enc_out, perp_out,
                a0v, a0i, a1v, a1i, a2v, a2i, sel, ssd, counts, bsqc):
    m = pl.program_id(0)
    e = pl.program_id(1)
    nm = pl.num_programs(0) - 1
    ne = pl.num_programs(1)

    liota = lax.broadcasted_iota(jnp.int32, (_TM, _TE), 1).astype(jnp.float32)
    e_off = (e * _TE).astype(jnp.float32)
    inf = jnp.float32(jnp.inf)
    bigf = jnp.float32(2.0 ** 24)

    oh = jnp.where(liota == sel[...] - e_off, 1.0, 0.0).astype(jnp.float32)
    enc_out[...] = oh
    col = jnp.sum(oh, axis=0, keepdims=True)

    @pl.when(m == 1)
    def _cinit():
        counts[pl.ds(e, 1), :] = col

    @pl.when(m > 1)
    def _cacc():
        counts[pl.ds(e, 1), :] = counts[pl.ds(e, 1), :] + col

    def lexpair(dist, mask=None):
        if mask is None:
            dm = dist
        else:
            dm = jnp.where(mask, dist, inf)
        tv = jnp.min(dm, axis=1, keepdims=True)
        pen = (dm - tv) * jnp.float32(1e12) + liota
        ti = jnp.min(pen, axis=1, keepdims=True)
        return tv, ti + e_off

    def merge(vref, iref, tv, ti):
        pv = vref[...]
        pi = iref[...]
        t = tv < pv
        vref[...] = jnp.where(t, tv, pv)
        iref[...] = jnp.where(t, ti, pi)

    @pl.when(m < nm)
    def _dist_work():
        a = flat_ref[...]
        b = embed_ref[...]
        asq = fsq_ref[...]

        @pl.when(m == 0)
        def _bsq0():
            bsqc[pl.ds(e, 1), :] = jnp.sum(b * b, axis=1)[None, :]

        bsq = bsqc[pl.ds(e, 1), :]
        mm = lax.dot_general(
            a.astype(jnp.bfloat16), b.astype(jnp.bfloat16),
            dimension_numbers=(((1,), (1,)), ((), ())),
            preferred_element_type=jnp.float32)
        dist = (asq + bsq) - 2.0 * mm

        @pl.when(e == 0)
        def _t0():
            tv, ti = lexpair(dist)
            a0v[...] = tv
            a0i[...] = ti
            a1v[...] = jnp.full((_TM, 1), inf)
            a1i[...] = jnp.zeros((_TM, 1), jnp.float32)
            a2v[...] = jnp.full((_TM, 1), inf)
            a2i[...] = jnp.zeros((_TM, 1), jnp.float32)

        @pl.when(e == 1)
        def _t1():
            tv, ti = lexpair(dist)
            merge(a0v, a0i, tv, ti)

        @pl.when(e == 2)
        def _t2():
            mk = liota < jnp.float32(_B1 - 2 * _TE)
            tv, ti = lexpair(dist, mk)
            merge(a0v, a0i, tv, ti)
            tv, ti = lexpair(dist, ~mk)
            merge(a1v, a1i, tv, ti)

        @pl.when((e == 3) | (e == 4))
        def _t34():
            tv, ti = lexpair(dist)
            merge(a1v, a1i, tv, ti)

        @pl.when(e == 5)
        def _t5():
            mk = liota < jnp.float32(_B2 - 5 * _TE)
            tv, ti = lexpair(dist, mk)
            merge(a1v, a1i, tv, ti)
            tv, ti = lexpair(dist, ~mk)
            merge(a2v, a2i, tv, ti)

        @pl.when((e == 6) | (e == 7))
        def _t67():
            tv, ti = lexpair(dist)
            merge(a2v, a2i, tv, ti)

        @pl.when(e == ne - 1)
        def _fin():
            c0v = a0v[...]; c0i = a0i[...]
            c1v = a1v[...]; c1i = a1i[...]
            c2v = a2v[...]; c2i = a2i[...]
            rv = c0v.astype(jnp.bfloat16).astype(jnp.float32)
            t1 = c1v < rv
            si = jnp.where(t1, c1i, c0i)
            sv = jnp.where(t1, c1v, c0v)
            rv2 = jnp.where(t1, c1v, rv).astype(jnp.bfloat16)
            rv2 = rv2.astype(jnp.float32)
            t2 = c2v < rv2
            si = jnp.where(t2, c2i, si)
            sv = jnp.where(t2, c2v, sv)
            idx_out[...] = si.astype(jnp.int32)
            sel[...] = si
            part = jnp.sum(sv)
            prev = jnp.where(m == 0, 0.0, ssd[0])
            ssd[0] = prev + part

    @pl.when((m == nm) & (e == ne - 1))
    def _final_scalars():
        n = jnp.float32(nm * _TM * _EMBEDDING_DIM)
        loss_out[...] = jnp.full(
            (1, 1), (1.0 + _COMMITMENT_COST) * ssd[0] / n, jnp.float32)
        total = jnp.float32(nm * _TM)
        p = counts[...] * (1.0 / total)
        s = jnp.sum(p * jnp.log(p + 1e-10))
        perp_out[...] = jnp.full((1, 1), jnp.exp(-s), jnp.float32)


def _sc_gather(embed, idx_flat, n_rows):
    d = embed.shape[1]
    nw = 32
    bpw = n_rows // nw
    ch = 128
    nch = bpw // ch
    mesh = plsc.VectorSubcoreMesh(core_axis_name="c", subcore_axis_name="s")

    @functools.partial(
        pl.kernel, mesh=mesh,
        out_type=jax.ShapeDtypeStruct((n_rows, d), jnp.float32),
        scratch_types=[
            pltpu.VMEM((bpw,), jnp.int32),
            pltpu.VMEM((ch, d), jnp.float32),
            pltpu.VMEM((ch, d), jnp.float32),
            pltpu.SemaphoreType.DMA,
            pltpu.SemaphoreType.DMA,
        ],
    )
    def gather(embed_hbm, idx_hbm, out_hbm, idx_v, rows0, rows1, sem0, sem1):
        wid = lax.axis_index("s") * 2 + lax.axis_index("c")
        base = wid * bpw
        pltpu.sync_copy(idx_hbm.at[pl.ds(base, bpw)], idx_v)
        bufs = (rows0, rows1)
        sems = (sem0, sem1)
        copies = [None, None]
        for c in range(nch):
            s = c % 2
            copies[s] = pltpu.async_copy(
                embed_hbm.at[idx_v.at[pl.ds(c * ch, ch)]], bufs[s], sems[s])
            if c > 0:
                prev = (c - 1) % 2
                copies[prev].wait()
                pltpu.sync_copy(bufs[prev],
                                out_hbm.at[pl.ds(base + (c - 1) * ch, ch)])
        last = (nch - 1) % 2
        copies[last].wait()
        pltpu.sync_copy(bufs[last],
                        out_hbm.at[pl.ds(base + (nch - 1) * ch, ch)])

    return gather(embed, idx_flat)


def kernel(inputs, embed):
    b, c, l = inputs.shape
    n = b * l
    v, d = embed.shape

    flat = jax.lax.optimization_barrier(
        jnp.transpose(inputs, (0, 2, 1)).reshape(n, c))
    fsq = jnp.sum(flat * flat, axis=1).reshape(n, 1)

    nm = n // _TM
    mlast = nm - 1
    grid = (nm + 1, v // _TE)
    idx2d, loss, enc, perp = pl.pallas_call(
        _fused_body,
        grid=grid,
        in_specs=[
            pl.BlockSpec((_TM, d), lambda m, e: (jnp.minimum(m, mlast), 0)),
            pl.BlockSpec((_TM, 1), lambda m, e: (jnp.minimum(m, mlast), 0)),
            pl.BlockSpec((_TE, d), lambda m, e: (e, 0)),
        ],
        out_specs=[
            pl.BlockSpec((_TM, 1), lambda m, e: (jnp.minimum(m, mlast), 0)),
            pl.BlockSpec((1, 1), lambda m, e: (0, 0)),
            pl.BlockSpec((_TM, _TE), lambda m, e: (jnp.maximum(m - 1, 0), e)),
            pl.BlockSpec((1, 1), lambda m, e: (0, 0)),
        ],
        out_shape=[
            jax.ShapeDtypeStruct((n, 1), jnp.int32),
            jax.ShapeDtypeStruct((1, 1), jnp.float32),
            jax.ShapeDtypeStruct((n, v), jnp.float32),
            jax.ShapeDtypeStruct((1, 1), jnp.float32),
        ],
        scratch_shapes=[
            pltpu.VMEM((_TM, 1), jnp.float32),
            pltpu.VMEM((_TM, 1), jnp.float32),
            pltpu.VMEM((_TM, 1), jnp.float32),
            pltpu.VMEM((_TM, 1), jnp.float32),
            pltpu.VMEM((_TM, 1), jnp.float32),
            pltpu.VMEM((_TM, 1), jnp.float32),
            pltpu.VMEM((_TM, 1), jnp.float32),
            pltpu.SMEM((1,), jnp.float32),
            pltpu.VMEM((v // _TE, _TE), jnp.float32),
            pltpu.VMEM((v // _TE, _TE), jnp.float32),
        ],
    )(flat, fsq, embed)

    qflat = _sc_gather(embed, idx2d.reshape(n), n)

    quantized_st = jnp.transpose(qflat.reshape(b, l, c), (0, 2, 1))
    return (loss.reshape(()), quantized_st, perp.reshape(()), embed,
            idx2d.reshape(b, l), enc)


# speedup vs baseline: 1.0291x; 1.0291x over previous
import functools

import jax
import jax.numpy as jnp
from jax import lax
from jax.experimental import pallas as pl
from jax.experimental.pallas import tpu as pltpu
from jax.experimental.pallas import tpu_sc as plsc

_NUM_EMBEDDINGS = 8192
_EMBEDDING_DIM = 256
_COMMITMENT_COST = 0.25

_TM = 1024
_TE = 1024
_B1 = 2736
_B2 = 5472


def _fused_body(flat_ref, fsq_ref, embed_ref,
                idx_out, loss_out, enc_out, perp_out,
                a0v, a0i, a1v, a1i, a2v, a2i, sel, ssd, counts):
    m = pl.program_id(0)
    e = pl.program_id(1)
    nm = pl.num_programs(0) - 1
    ne = pl.num_programs(1)

    liota = lax.broadcasted_iota(jnp.int32, (_TM, _TE), 1).astype(jnp.float32)
    e_off = (e * _TE).astype(jnp.float32)
    inf = jnp.float32(jnp.inf)
    bigf = jnp.float32(2.0 ** 24)

    oh = jnp.where(liota == sel[...] - e_off, 1.0, 0.0).astype(jnp.float32)
    enc_out[...] = oh
    col = jnp.sum(oh, axis=0, keepdims=True)

    @pl.when(m == 1)
    def _cinit():
        counts[pl.ds(e, 1), :] = col

    @pl.when(m > 1)
    def _cacc():
        counts[pl.ds(e, 1), :] = counts[pl.ds(e, 1), :] + col

    def lexpair(dist, mask=None):
        if mask is None:
            dm = dist
        else:
            dm = jnp.where(mask, dist, inf)
        tv = jnp.min(dm, axis=1, keepdims=True)
        pen = (dm - tv) * jnp.float32(1e12) + liota
        ti = jnp.min(pen, axis=1, keepdims=True)
        return tv, ti + e_off

    def merge(vref, iref, tv, ti):
        pv = vref[...]
        pi = iref[...]
        t = tv < pv
        vref[...] = jnp.where(t, tv, pv)
        iref[...] = jnp.where(t, ti, pi)

    @pl.when(m < nm)
    def _dist_work():
        a = flat_ref[...]
        b = embed_ref[...]
        asq = fsq_ref[...]
        bsq = jnp.sum(b * b, axis=1)[None, :]
        mm = lax.dot_general(
            a.astype(jnp.bfloat16), b.astype(jnp.bfloat16),
            dimension_numbers=(((1,), (1,)), ((), ())),
            preferred_element_type=jnp.float32)
        dist = (asq + bsq) - 2.0 * mm

        @pl.when(e == 0)
        def _t0():
            tv, ti = lexpair(dist)
            a0v[...] = tv
            a0i[...] = ti
            a1v[...] = jnp.full((_TM, 1), inf)
            a1i[...] = jnp.zeros((_TM, 1), jnp.float32)
            a2v[...] = jnp.full((_TM, 1), inf)
            a2i[...] = jnp.zeros((_TM, 1), jnp.float32)

        @pl.when(e == 1)
        def _t1():
            tv, ti = lexpair(dist)
            merge(a0v, a0i, tv, ti)

        @pl.when(e == 2)
        def _t2():
            mk = liota < jnp.float32(_B1 - 2 * _TE)
            tv, ti = lexpair(dist, mk)
            merge(a0v, a0i, tv, ti)
            tv, ti = lexpair(dist, ~mk)
            merge(a1v, a1i, tv, ti)

        @pl.when((e == 3) | (e == 4))
        def _t34():
            tv, ti = lexpair(dist)
            merge(a1v, a1i, tv, ti)

        @pl.when(e == 5)
        def _t5():
            mk = liota < jnp.float32(_B2 - 5 * _TE)
            tv, ti = lexpair(dist, mk)
            merge(a1v, a1i, tv, ti)
            tv, ti = lexpair(dist, ~mk)
            merge(a2v, a2i, tv, ti)

        @pl.when((e == 6) | (e == 7))
        def _t67():
            tv, ti = lexpair(dist)
            merge(a2v, a2i, tv, ti)

        @pl.when(e == ne - 1)
        def _fin():
            c0v = a0v[...]; c0i = a0i[...]
            c1v = a1v[...]; c1i = a1i[...]
            c2v = a2v[...]; c2i = a2i[...]
            rv = c0v.astype(jnp.bfloat16).astype(jnp.float32)
            t1 = c1v < rv
            si = jnp.where(t1, c1i, c0i)
            sv = jnp.where(t1, c1v, c0v)
            rv2 = jnp.where(t1, c1v, rv).astype(jnp.bfloat16)
            rv2 = rv2.astype(jnp.float32)
            t2 = c2v < rv2
            si = jnp.where(t2, c2i, si)
            sv = jnp.where(t2, c2v, sv)
            idx_out[...] = si.astype(jnp.int32)
            sel[...] = si
            part = jnp.sum(sv)
            prev = jnp.where(m == 0, 0.0, ssd[0])
            ssd[0] = prev + part

    @pl.when((m == nm) & (e == ne - 1))
    def _final_scalars():
        n = jnp.float32(nm * _TM * _EMBEDDING_DIM)
        loss_out[...] = jnp.full(
            (1, 1), (1.0 + _COMMITMENT_COST) * ssd[0] / n, jnp.float32)
        total = jnp.float32(nm * _TM)
        p = counts[...] * (1.0 / total)
        s = jnp.sum(p * jnp.log(p + 1e-10))
        perp_out[...] = jnp.full((1, 1), jnp.exp(-s), jnp.float32)


def _sc_gather(embed, idx_flat, n_rows):
    d = embed.shape[1]
    nw = 32
    bpw = n_rows // nw
    ch = 128
    nch = bpw // ch
    mesh = plsc.VectorSubcoreMesh(core_axis_name="c", subcore_axis_name="s")

    @functools.partial(
        pl.kernel, mesh=mesh,
        out_type=jax.ShapeDtypeStruct((n_rows, d), jnp.float32),
        scratch_types=[
            pltpu.VMEM((bpw,), jnp.int32),
            pltpu.VMEM((ch, d), jnp.float32),
            pltpu.VMEM((ch, d), jnp.float32),
            pltpu.SemaphoreType.DMA,
            pltpu.SemaphoreType.DMA,
        ],
    )
    def gather(embed_hbm, idx_hbm, out_hbm, idx_v, rows0, rows1, sem0, sem1):
        wid = lax.axis_index("s") * 2 + lax.axis_index("c")
        base = wid * bpw
        pltpu.sync_copy(idx_hbm.at[pl.ds(base, bpw)], idx_v)
        bufs = (rows0, rows1)
        sems = (sem0, sem1)
        copies = [None, None]
        for c in range(nch):
            s = c % 2
            copies[s] = pltpu.async_copy(
                embed_hbm.at[idx_v.at[pl.ds(c * ch, ch)]], bufs[s], sems[s])
            if c > 0:
                prev = (c - 1) % 2
                copies[prev].wait()
                pltpu.sync_copy(bufs[prev],
                                out_hbm.at[pl.ds(base + (c - 1) * ch, ch)])
        last = (nch - 1) % 2
        copies[last].wait()
        pltpu.sync_copy(bufs[last],
                        out_hbm.at[pl.ds(base + (nch - 1) * ch, ch)])

    return gather(embed, idx_flat)


def kernel(inputs, embed):
    b, c, l = inputs.shape
    n = b * l
    v, d = embed.shape

    flat = jax.lax.optimization_barrier(
        jnp.transpose(inputs, (0, 2, 1)).reshape(n, c))
    fsq = jnp.sum(flat * flat, axis=1).reshape(n, 1)

    nm = n // _TM
    mlast = nm - 1
    grid = (nm + 1, v // _TE)
    idx2d, loss, enc, perp = pl.pallas_call(
        _fused_body,
        grid=grid,
        in_specs=[
            pl.BlockSpec((_TM, d), lambda m, e: (jnp.minimum(m, mlast), 0)),
            pl.BlockSpec((_TM, 1), lambda m, e: (jnp.minimum(m, mlast), 0)),
            pl.BlockSpec((_TE, d), lambda m, e: (e, 0)),
        ],
        out_specs=[
            pl.BlockSpec((_TM, 1), lambda m, e: (jnp.minimum(m, mlast), 0)),
            pl.BlockSpec((1, 1), lambda m, e: (0, 0)),
            pl.BlockSpec((_TM, _TE), lambda m, e: (jnp.maximum(m - 1, 0), e)),
            pl.BlockSpec((1, 1), lambda m, e: (0, 0)),
        ],
        out_shape=[
            jax.ShapeDtypeStruct((n, 1), jnp.int32),
            jax.ShapeDtypeStruct((1, 1), jnp.float32),
            jax.ShapeDtypeStruct((n, v), jnp.float32),
            jax.ShapeDtypeStruct((1, 1), jnp.float32),
        ],
        scratch_shapes=[
            pltpu.VMEM((_TM, 1), jnp.float32),
            pltpu.VMEM((_TM, 1), jnp.float32),
            pltpu.VMEM((_TM, 1), jnp.float32),
            pltpu.VMEM((_TM, 1), jnp.float32),
            pltpu.VMEM((_TM, 1), jnp.float32),
            pltpu.VMEM((_TM, 1), jnp.float32),
            pltpu.VMEM((_TM, 1), jnp.float32),
            pltpu.SMEM((1,), jnp.float32),
            pltpu.VMEM((v // _TE, _TE), jnp.float32),
        ],
    )(flat, fsq, embed)

    qflat = _sc_gather(embed, idx2d.reshape(n), n)

    quantized_st = jnp.transpose(qflat.reshape(b, l, c), (0, 2, 1))
    return (loss.reshape(()), quantized_st, perp.reshape(()), embed,
            idx2d.reshape(b, l), enc)


# speedup vs baseline: 1.0435x; 1.0141x over previous
import functools

import jax
import jax.numpy as jnp
from jax import lax
from jax.experimental import pallas as pl
from jax.experimental.pallas import tpu as pltpu
from jax.experimental.pallas import tpu_sc as plsc

_NUM_EMBEDDINGS = 8192
_EMBEDDING_DIM = 256
_COMMITMENT_COST = 0.25

_TM = 1024
_TE = 1024
_B1 = 2736
_B2 = 5472


def _fused_body(flat_ref, fsq_ref, embed_ref,
                idx_out, loss_out, enc_out, perp_out,
                a0v, a0i, a1v, a1i, a2v, a2i, sel, ssd, counts):
    m = pl.program_id(0)
    e = pl.program_id(1)
    nm = pl.num_programs(0) - 1
    ne = pl.num_programs(1)

    liota = lax.broadcasted_iota(jnp.int32, (_TM, _TE), 1).astype(jnp.float32)
    e_off = (e * _TE).astype(jnp.float32)
    inf = jnp.float32(jnp.inf)
    bigf = jnp.float32(2.0 ** 24)

    oh = jnp.where(liota == sel[...] - e_off, 1.0, 0.0).astype(jnp.float32)
    enc_out[...] = oh
    col = jnp.sum(oh, axis=0, keepdims=True)

    @pl.when(m == 1)
    def _cinit():
        counts[pl.ds(e, 1), :] = col

    @pl.when(m > 1)
    def _cacc():
        counts[pl.ds(e, 1), :] = counts[pl.ds(e, 1), :] + col

    def lexpair(dist, mask=None):
        if mask is None:
            tv = jnp.min(dist, axis=1, keepdims=True)
            ti = jnp.min(jnp.where(dist == tv, liota, bigf),
                         axis=1, keepdims=True)
        else:
            dm = jnp.where(mask, dist, inf)
            tv = jnp.min(dm, axis=1, keepdims=True)
            ti = jnp.min(jnp.where(mask & (dist == tv), liota, bigf),
                         axis=1, keepdims=True)
        return tv, ti + e_off

    def merge(vref, iref, tv, ti):
        pv = vref[...]
        pi = iref[...]
        t = tv < pv
        vref[...] = jnp.where(t, tv, pv)
        iref[...] = jnp.where(t, ti, pi)

    @pl.when(m < nm)
    def _dist_work():
        a = flat_ref[...]
        b = embed_ref[...]
        asq = fsq_ref[...]
        bsq = jnp.sum(b * b, axis=1)[None, :]
        mm = lax.dot_general(
            a.astype(jnp.bfloat16), b.astype(jnp.bfloat16),
            dimension_numbers=(((1,), (1,)), ((), ())),
            preferred_element_type=jnp.float32)
        dist = (asq + bsq) - 2.0 * mm

        @pl.when(e == 0)
        def _t0():
            tv, ti = lexpair(dist)
            a0v[...] = tv
            a0i[...] = ti
            a1v[...] = jnp.full((_TM, 1), inf)
            a1i[...] = jnp.zeros((_TM, 1), jnp.float32)
            a2v[...] = jnp.full((_TM, 1), inf)
            a2i[...] = jnp.zeros((_TM, 1), jnp.float32)

        @pl.when(e == 1)
        def _t1():
            tv, ti = lexpair(dist)
            merge(a0v, a0i, tv, ti)

        @pl.when(e == 2)
        def _t2():
            mk = liota < jnp.float32(_B1 - 2 * _TE)
            tv, ti = lexpair(dist, mk)
            merge(a0v, a0i, tv, ti)
            tv, ti = lexpair(dist, ~mk)
            merge(a1v, a1i, tv, ti)

        @pl.when((e == 3) | (e == 4))
        def _t34():
            tv, ti = lexpair(dist)
            merge(a1v, a1i, tv, ti)

        @pl.when(e == 5)
        def _t5():
            mk = liota < jnp.float32(_B2 - 5 * _TE)
            tv, ti = lexpair(dist, mk)
            merge(a1v, a1i, tv, ti)
            tv, ti = lexpair(dist, ~mk)
            merge(a2v, a2i, tv, ti)

        @pl.when((e == 6) | (e == 7))
        def _t67():
            tv, ti = lexpair(dist)
            merge(a2v, a2i, tv, ti)

        @pl.when(e == ne - 1)
        def _fin():
            c0v = a0v[...]; c0i = a0i[...]
            c1v = a1v[...]; c1i = a1i[...]
            c2v = a2v[...]; c2i = a2i[...]
            rv = c0v.astype(jnp.bfloat16).astype(jnp.float32)
            t1 = c1v < rv
            si = jnp.where(t1, c1i, c0i)
            sv = jnp.where(t1, c1v, c0v)
            rv2 = jnp.where(t1, c1v, rv).astype(jnp.bfloat16)
            rv2 = rv2.astype(jnp.float32)
            t2 = c2v < rv2
            si = jnp.where(t2, c2i, si)
            sv = jnp.where(t2, c2v, sv)
            idx_out[...] = si.astype(jnp.int32)
            sel[...] = si
            part = jnp.sum(sv)
            prev = jnp.where(m == 0, 0.0, ssd[0])
            ssd[0] = prev + part

    @pl.when((m == nm) & (e == ne - 1))
    def _final_scalars():
        n = jnp.float32(nm * _TM * _EMBEDDING_DIM)
        loss_out[...] = jnp.full(
            (1, 1), (1.0 + _COMMITMENT_COST) * ssd[0] / n, jnp.float32)
        total = jnp.float32(nm * _TM)
        p = counts[...] * (1.0 / total)
        s = jnp.sum(p * jnp.log(p + 1e-10))
        perp_out[...] = jnp.full((1, 1), jnp.exp(-s), jnp.float32)


def _sc_gather(embed, idx_flat, n_rows):
    d = embed.shape[1]
    nw = 32
    bpw = n_rows // nw
    ch = 128
    nch = bpw // ch
    mesh = plsc.VectorSubcoreMesh(core_axis_name="c", subcore_axis_name="s")

    @functools.partial(
        pl.kernel, mesh=mesh,
        out_type=jax.ShapeDtypeStruct((n_rows, d), jnp.float32),
        scratch_types=[
            pltpu.VMEM((bpw,), jnp.int32),
            pltpu.VMEM((ch, d), jnp.float32),
            pltpu.VMEM((ch, d), jnp.float32),
            pltpu.SemaphoreType.DMA,
            pltpu.SemaphoreType.DMA,
        ],
    )
    def gather(embed_hbm, idx_hbm, out_hbm, idx_v, rows0, rows1, sem0, sem1):
        wid = lax.axis_index("s") * 2 + lax.axis_index("c")
        base = wid * bpw
        pltpu.sync_copy(idx_hbm.at[pl.ds(base, bpw)], idx_v)
        bufs = (rows0, rows1)
        sems = (sem0, sem1)
        copies = [None, None]
        for c in range(nch):
            s = c % 2
            copies[s] = pltpu.async_copy(
                embed_hbm.at[idx_v.at[pl.ds(c * ch, ch)]], bufs[s], sems[s])
            if c > 0:
                prev = (c - 1) % 2
                copies[prev].wait()
                pltpu.sync_copy(bufs[prev],
                                out_hbm.at[pl.ds(base + (c - 1) * ch, ch)])
        last = (nch - 1) % 2
        copies[last].wait()
        pltpu.sync_copy(bufs[last],
                        out_hbm.at[pl.ds(base + (nch - 1) * ch, ch)])

    return gather(embed, idx_flat)


def kernel(inputs, embed):
    b, c, l = inputs.shape
    n = b * l
    v, d = embed.shape

    flat = jax.lax.optimization_barrier(
        jnp.transpose(inputs, (0, 2, 1)).reshape(n, c))
    fsq = jnp.sum(flat * flat, axis=1).reshape(n, 1)

    nm = n // _TM
    mlast = nm - 1
    grid = (nm + 1, v // _TE)
    idx2d, loss, enc, perp = pl.pallas_call(
        _fused_body,
        grid=grid,
        in_specs=[
            pl.BlockSpec((_TM, d), lambda m, e: (jnp.minimum(m, mlast), 0)),
            pl.BlockSpec((_TM, 1), lambda m, e: (jnp.minimum(m, mlast), 0)),
            pl.BlockSpec((_TE, d), lambda m, e: (e, 0)),
        ],
        out_specs=[
            pl.BlockSpec((_TM, 1), lambda m, e: (jnp.minimum(m, mlast), 0)),
            pl.BlockSpec((1, 1), lambda m, e: (0, 0)),
            pl.BlockSpec((_TM, _TE), lambda m, e: (jnp.maximum(m - 1, 0), e)),
            pl.BlockSpec((1, 1), lambda m, e: (0, 0)),
        ],
        out_shape=[
            jax.ShapeDtypeStruct((n, 1), jnp.int32),
            jax.ShapeDtypeStruct((1, 1), jnp.float32),
            jax.ShapeDtypeStruct((n, v), jnp.float32),
            jax.ShapeDtypeStruct((1, 1), jnp.float32),
        ],
        scratch_shapes=[
            pltpu.VMEM((_TM, 1), jnp.float32),
            pltpu.VMEM((_TM, 1), jnp.float32),
            pltpu.VMEM((_TM, 1), jnp.float32),
            pltpu.VMEM((_TM, 1), jnp.float32),
            pltpu.VMEM((_TM, 1), jnp.float32),
            pltpu.VMEM((_TM, 1), jnp.float32),
            pltpu.VMEM((_TM, 1), jnp.float32),
            pltpu.SMEM((1,), jnp.float32),
            pltpu.VMEM((v // _TE, _TE), jnp.float32),
        ],
    )(flat, fsq, embed)

    qflat = _sc_gather(embed, idx2d.reshape(n), n)

    quantized_st = jnp.transpose(qflat.reshape(b, l, c), (0, 2, 1))
    return (loss.reshape(()), quantized_st, perp.reshape(()), embed,
            idx2d.reshape(b, l), enc)
